# baseline (speedup 1.0000x reference)
import functools

import jax
import jax.numpy as jnp
from jax import lax
from jax.experimental import pallas as pl
from jax.experimental.pallas import tpu as pltpu
from jax.experimental.pallas import tpu_sc as plsc

N = 10000
NP = 10240
E = 320000
EPAD = 327680
D_IN = 128
H = 256
HH = 128
S = 11
EPS = 1e-5
BNS = 1.0 / (1.0 + EPS) ** 0.5
K = 128
RB = NP // 16
R = 1024
NB = NP // R
NG = EPAD // 16 // K
NBUF = 4
NGD = EPAD // 32 // K

_mesh = plsc.VectorSubcoreMesh(core_axis_name="c", subcore_axis_name="s")


@functools.partial(
    pl.kernel, mesh=_mesh,
    out_type=jax.ShapeDtypeStruct((2 * NP,), jnp.float32),
    scratch_types=[
        pltpu.VMEM((NGD, K), jnp.int32),
        pltpu.VMEM((K,), jnp.float32),
        pltpu.VMEM((RB,), jnp.float32),
        pltpu.VMEM_SHARED((NP,), jnp.float32),
    ],
)
def _deg(dst3_hbm, out_hbm, didx, ones_v, zbuf, acc):
    c = lax.axis_index("c")
    s = lax.axis_index("s")
    w = c * 16 + s

    def fill_ones(i, _):
        ones_v[pl.ds(i * 16, 16)] = jnp.ones((16,), jnp.float32)
        return 0

    lax.fori_loop(0, K // 16, fill_ones, 0)

    def fill_zero(i, _):
        zbuf[pl.ds(i * 16, 16)] = jnp.zeros((16,), jnp.float32)
        return 0

    lax.fori_loop(0, RB // 16, fill_zero, 0)
    pltpu.sync_copy(dst3_hbm.at[w], didx)
    pltpu.sync_copy(zbuf, acc.at[pl.ds(s * RB, RB)])
    plsc.subcore_barrier()

    def chunk(g, _):
        pltpu.sync_copy(ones_v, acc.at[didx.at[g]], add=True)
        return 0

    lax.fori_loop(0, NGD, chunk, 0)
    plsc.subcore_barrier()
    pltpu.sync_copy(acc.at[pl.ds(s * RB, RB)],
                    out_hbm.at[pl.ds(c * NP + s * RB, RB)])


@functools.partial(
    pl.kernel, mesh=_mesh,
    out_type=jax.ShapeDtypeStruct((2 * NP, HH), jnp.float32),
    scratch_types=[
        pltpu.VMEM((2, 8, K), jnp.int32),
        pltpu.VMEM((2, 8, K), jnp.int32),
        pltpu.VMEM((K, HH), jnp.float32),
        pltpu.VMEM((K, HH), jnp.float32),
        pltpu.SemaphoreType.DMA,
        pltpu.SemaphoreType.DMA,
        pltpu.SemaphoreType.DMA,
        pltpu.SemaphoreType.DMA,
        pltpu.SemaphoreType.DMA,
        pltpu.SemaphoreType.DMA,
        pltpu.SemaphoreType.DMA,
        pltpu.SemaphoreType.DMA,
        pltpu.VMEM_SHARED((NP, HH), jnp.float32),
    ],
)
def _agg(hs_hbm, srcs4_hbm, dst4_hbm, out_hbm, sidx, didx, r0, r1,
         ss0, ss1, ds0, ds1, gs0, gs1, cs0, cs1, acc):
    c = lax.axis_index("c")
    s = lax.axis_index("s")
    w = c * 16 + s
    rows = [r0, r1]
    ssem = [ss0, ss1]
    dsem = [ds0, ds1]
    gsem = [gs0, gs1]
    csem = [cs0, cs1]
    NGRP = NG // 8

    def grp_start(h, hb):
        pltpu.make_async_copy(srcs4_hbm.at[w].at[h], sidx.at[hb],
                              ssem[hb]).start()
        pltpu.make_async_copy(dst4_hbm.at[s].at[h], didx.at[hb],
                              dsem[hb]).start()

    def grp_wait(h, hb):
        pltpu.make_async_copy(srcs4_hbm.at[w].at[h], sidx.at[hb],
                              ssem[hb]).wait()
        pltpu.make_async_copy(dst4_hbm.at[s].at[h], didx.at[hb],
                              dsem[hb]).wait()

    def gat_start(hb, k, b):
        pltpu.make_async_copy(hs_hbm.at[sidx.at[hb].at[k]], rows[b],
                              gsem[b]).start()

    def gat_wait(hb, k, b):
        pltpu.make_async_copy(hs_hbm.at[sidx.at[hb].at[k]], rows[b],
                              gsem[b]).wait()

    def sc_start(hb, k, b):
        pltpu.async_copy(rows[b], acc.at[didx.at[hb].at[k]], csem[b],
                         add=True)

    def sc_wait(hb, k, b):
        pltpu.make_async_copy(rows[b], acc.at[didx.at[hb].at[k]],
                              csem[b]).wait()

    pltpu.sync_copy(hs_hbm.at[pl.ds(c * NP + s * RB, RB)],
                    acc.at[pl.ds(s * RB, RB)])
    grp_start(0, 0)
    plsc.subcore_barrier()

    def outer(h0, _):
        for p in range(2):
            h = h0 * 2 + p
            for k in range(8):
                b = k % 2
                if k >= 2:
                    sc_wait(p, k - 2, b)
                elif p == 1:
                    sc_wait(0, k + 6, b)
                else:
                    @pl.when(h0 >= 1)
                    def _():
                        sc_wait(1, k + 6, b)
                if k == 0:
                    grp_wait(h, p)
                gat_start(p, k, b)
                if k >= 1:
                    gat_wait(p, k - 1, 1 - b)
                    sc_start(p, k - 1, 1 - b)
                elif p == 1:
                    gat_wait(0, 7, 1 - b)
                    sc_start(0, 7, 1 - b)
                else:
                    @pl.when(h0 >= 1)
                    def _():
                        gat_wait(1, 7, 1 - b)
                        sc_start(1, 7, 1 - b)
                if k == 2:
                    if p == 0:
                        grp_start(h + 1, 1)
                    else:
                        @pl.when(h0 < NGRP // 2 - 1)
                        def _():
                            grp_start(h + 1, 0)
        return 0

    lax.fori_loop(0, NGRP // 2, outer, 0)
    gat_wait(1, 7, 1)
    sc_start(1, 7, 1)
    sc_wait(1, 6, 0)
    sc_wait(1, 7, 1)
    plsc.subcore_barrier()
    pltpu.sync_copy(acc.at[pl.ds(s * RB, RB)],
                    out_hbm.at[pl.ds(c * NP + s * RB, RB)])


def _tc1(x_ref, w_ref, deg_ref, out_ref):
    dinv = lax.rsqrt(deg_ref[0, :] + deg_ref[1, :] + 1.0)
    t = jnp.dot(x_ref[...], w_ref[...], preferred_element_type=jnp.float32)
    t = t * dinv[:, None]
    out_ref[0] = t[:, :HH]
    out_ref[1] = t[:, HH:]


def _tc2(a_ref, deg_ref, b_ref, g_ref, be_ref, w_ref, out_ref):
    dinv = lax.rsqrt(deg_ref[0, :] + deg_ref[1, :] + 1.0)
    a = jnp.concatenate([a_ref[0], a_ref[1]], axis=1)
    h = a * dinv[:, None] + b_ref[...]
    h = jnp.maximum(h * (g_ref[...] * BNS) + be_ref[...], 0.0)
    t = jnp.dot(h.astype(jnp.bfloat16), w_ref[...],
                preferred_element_type=jnp.float32)
    t = t * dinv[:, None]
    out_ref[0] = t[:, :HH]
    out_ref[1] = t[:, HH:]


def _tc3(a_ref, deg_ref, b_ref, g_ref, be_ref, w_ref, fb_ref, sec_ref,
         w2_ref, b2_ref, hw1_ref, hb1_ref, hw2_ref, hb2_ref,
         out_ref, tsum_ref, cnt_ref):
    i = pl.program_id(0)
    dinv = lax.rsqrt(deg_ref[0, :] + deg_ref[1, :] + 1.0)
    a = jnp.concatenate([a_ref[0], a_ref[1]], axis=1)
    h = a * dinv[:, None] + b_ref[...]
    h = jnp.maximum(h * (g_ref[...] * BNS) + be_ref[...], 0.0)
    t = jnp.maximum(
        jnp.dot(h.astype(jnp.bfloat16), w_ref[...],
                preferred_element_type=jnp.float32)
        + fb_ref[...], 0.0)
    iot = lax.broadcasted_iota(jnp.int32, (1, S), 1)
    oh = (sec_ref[...] == iot).astype(jnp.float32)
    ts = lax.dot_general(oh, t, (((0,), (0,)), ((), ())),
                         preferred_element_type=jnp.float32)
    cs = jnp.sum(oh, axis=0)[:, None]

    @pl.when(i == 0)
    def _():
        tsum_ref[...] = ts
        cnt_ref[...] = cs

    @pl.when(i > 0)
    def _():
        tsum_ref[...] += ts
        cnt_ref[...] += cs

    @pl.when(i == NB - 1)
    def _():
        cnt = cnt_ref[...]
        meant = tsum_ref[...] / jnp.maximum(cnt, 1.0)
        se = jnp.dot(meant, w2_ref[...], preferred_element_type=jnp.float32)
        se = se + b2_ref[...]
        se = jnp.where(cnt > 0.0, se, 0.0)
        prows = []
        for k in range(S):
            v = jnp.dot(se[k:k + 1, :], hw1_ref[k],
                        preferred_element_type=jnp.float32)
            v = jnp.maximum(v + hb1_ref[k:k + 1, :], 0.0)
            p = (jnp.sum(v * hw2_ref[k], axis=1, keepdims=True)
                 + hb2_ref[k:k + 1, :])
            prows.append(p)
        out_ref[...] = jnp.concatenate(prows, axis=0)


def kernel(x, edge_index, sectors, W0, b0, W1, b1, g0, be0, g1, be1,
           fcW1, fcb1, fcW2, fcb2, HW1, Hb1, HW2, Hb2):
    f32 = jnp.float32
    src, dst = edge_index[0], edge_index[1]
    padn = NP - N
    x_pad = jnp.pad(x, ((0, padn), (0, 0)))
    sec_pad = jnp.pad(sectors, (0, padn), constant_values=S)[:, None]
    pade = EPAD - E
    filler = N + (jnp.arange(pade, dtype=jnp.int32) % padn)
    src_p = jnp.concatenate([src, filler])
    dst_p = jnp.concatenate([dst, filler])
    srcs3 = jnp.concatenate([src_p, src_p + NP]).reshape(32, NG // 8, 8, K)
    dst3 = dst_p.reshape(16, NG // 8, 8, K)
    dst3d = dst_p.reshape(32, NGD, K)

    degpair = _deg(dst3d).reshape(2, NP)

    hs0 = pl.pallas_call(
        _tc1, grid=(NB,),
        in_specs=[pl.BlockSpec((R, D_IN), lambda i: (i, 0)),
                  pl.BlockSpec((D_IN, H), lambda i: (0, 0)),
                  pl.BlockSpec((2, R), lambda i: (0, i))],
        out_specs=pl.BlockSpec((2, R, HH), lambda i: (0, i, 0)),
        out_shape=jax.ShapeDtypeStruct((2, NP, HH), f32),
    )(x_pad.astype(jnp.bfloat16), W0.astype(jnp.bfloat16), degpair)

    agg0 = _agg(hs0.reshape(2 * NP, HH), srcs3, dst3).reshape(2, NP, HH)

    hs1 = pl.pallas_call(
        _tc2, grid=(NB,),
        in_specs=[pl.BlockSpec((2, R, HH), lambda i: (0, i, 0)),
                  pl.BlockSpec((2, R), lambda i: (0, i)),
                  pl.BlockSpec((1, H), lambda i: (0, 0)),
                  pl.BlockSpec((1, H), lambda i: (0, 0)),
                  pl.BlockSpec((1, H), lambda i: (0, 0)),
                  pl.BlockSpec((H, H), lambda i: (0, 0))],
        out_specs=pl.BlockSpec((2, R, HH), lambda i: (0, i, 0)),
        out_shape=jax.ShapeDtypeStruct((2, NP, HH), f32),
    )(agg0, degpair, b0[None, :], g0[None, :], be0[None, :],
      W1.astype(jnp.bfloat16))

    agg1 = _agg(hs1.reshape(2 * NP, HH), srcs3, dst3).reshape(2, NP, HH)

    preds = pl.pallas_call(
        _tc3, grid=(NB,),
        in_specs=[pl.BlockSpec((2, R, HH), lambda i: (0, i, 0)),
                  pl.BlockSpec((2, R), lambda i: (0, i)),
                  pl.BlockSpec((1, H), lambda i: (0, 0)),
                  pl.BlockSpec((1, H), lambda i: (0, 0)),
                  pl.BlockSpec((1, H), lambda i: (0, 0)),
                  pl.BlockSpec((H, HH), lambda i: (0, 0)),
                  pl.BlockSpec((1, HH), lambda i: (0, 0)),
                  pl.BlockSpec((R, 1), lambda i: (i, 0)),
                  pl.BlockSpec((HH, H), lambda i: (0, 0)),
                  pl.BlockSpec((1, H), lambda i: (0, 0)),
                  pl.BlockSpec((S, H, HH), lambda i: (0, 0, 0)),
                  pl.BlockSpec((S, HH), lambda i: (0, 0)),
                  pl.BlockSpec((S, 1, HH), lambda i: (0, 0, 0)),
                  pl.BlockSpec((S, 1), lambda i: (0, 0))],
        out_specs=pl.BlockSpec((S, 1), lambda i: (0, 0)),
        out_shape=jax.ShapeDtypeStruct((S, 1), f32),
        scratch_shapes=[pltpu.VMEM((S, HH), f32), pltpu.VMEM((S, 1), f32)],
    )(agg1, degpair, b1[None, :], g1[None, :], be1[None, :],
      fcW1.astype(jnp.bfloat16),
      fcb1[None, :], sec_pad, fcW2, fcb2[None, :], HW1, Hb1,
      jnp.transpose(HW2, (0, 2, 1)), Hb2)
    return preds


# speedup vs baseline: 1.0006x; 1.0006x over previous
import functools

import jax
import jax.numpy as jnp
from jax import lax
from jax.experimental import pallas as pl
from jax.experimental.pallas import tpu as pltpu
from jax.experimental.pallas import tpu_sc as plsc

N = 10000
NP = 10240
E = 320000
EPAD = 327680
D_IN = 128
H = 256
HH = 128
S = 11
EPS = 1e-5
BNS = 1.0 / (1.0 + EPS) ** 0.5
K = 128
RB = NP // 16
R = 1024
NB = NP // R
NG = EPAD // 16 // K
NGD = EPAD // 32 // K

_mesh = plsc.VectorSubcoreMesh(core_axis_name="c", subcore_axis_name="s")


@functools.partial(
    pl.kernel, mesh=_mesh,
    out_type=jax.ShapeDtypeStruct((2 * NP,), jnp.float32),
    scratch_types=[
        pltpu.VMEM((NGD, K), jnp.int32),
        pltpu.VMEM((K,), jnp.float32),
        pltpu.VMEM((RB,), jnp.float32),
        pltpu.VMEM_SHARED((NP,), jnp.float32),
    ],
)
def _deg(dst3_hbm, out_hbm, didx, ones_v, zbuf, acc):
    c = lax.axis_index("c")
    s = lax.axis_index("s")
    w = c * 16 + s

    def fill_ones(i, _):
        ones_v[pl.ds(i * 16, 16)] = jnp.ones((16,), jnp.float32)
        return 0

    lax.fori_loop(0, K // 16, fill_ones, 0)

    def fill_zero(i, _):
        zbuf[pl.ds(i * 16, 16)] = jnp.zeros((16,), jnp.float32)
        return 0

    lax.fori_loop(0, RB // 16, fill_zero, 0)
    pltpu.sync_copy(dst3_hbm.at[w], didx)
    pltpu.sync_copy(zbuf, acc.at[pl.ds(s * RB, RB)])
    plsc.subcore_barrier()

    def chunk(g, _):
        pltpu.sync_copy(ones_v, acc.at[didx.at[g]], add=True)
        return 0

    lax.fori_loop(0, NGD, chunk, 0)
    plsc.subcore_barrier()
    pltpu.sync_copy(acc.at[pl.ds(s * RB, RB)],
                    out_hbm.at[pl.ds(c * NP + s * RB, RB)])


@functools.partial(
    pl.kernel, mesh=_mesh,
    out_type=jax.ShapeDtypeStruct((2 * NP, HH), jnp.float32),
    scratch_types=[
        pltpu.VMEM((2, 8, K), jnp.int32),
        pltpu.VMEM((2, 8, K), jnp.int32),
        pltpu.VMEM((K, HH), jnp.float32),
        pltpu.VMEM((K, HH), jnp.float32),
        pltpu.SemaphoreType.DMA,
        pltpu.SemaphoreType.DMA,
        pltpu.SemaphoreType.DMA,
        pltpu.SemaphoreType.DMA,
        pltpu.SemaphoreType.DMA,
        pltpu.SemaphoreType.DMA,
        pltpu.SemaphoreType.DMA,
        pltpu.SemaphoreType.DMA,
        pltpu.VMEM_SHARED((NP, HH), jnp.float32),
    ],
)
def _agg(hs_hbm, srcs4_hbm, dst4_hbm, out_hbm, sidx, didx, r0, r1,
         ss0, ss1, ds0, ds1, gs0, gs1, cs0, cs1, acc):
    c = lax.axis_index("c")
    s = lax.axis_index("s")
    w = c * 16 + s
    rows = [r0, r1]
    ssem = [ss0, ss1]
    dsem = [ds0, ds1]
    gsem = [gs0, gs1]
    csem = [cs0, cs1]
    NGRP = NG // 8

    def grp_start(h, hb):
        pltpu.make_async_copy(srcs4_hbm.at[w].at[h], sidx.at[hb],
                              ssem[hb]).start()
        pltpu.make_async_copy(dst4_hbm.at[s].at[h], didx.at[hb],
                              dsem[hb]).start()

    def grp_wait(h, hb):
        pltpu.make_async_copy(srcs4_hbm.at[w].at[h], sidx.at[hb],
                              ssem[hb]).wait()
        pltpu.make_async_copy(dst4_hbm.at[s].at[h], didx.at[hb],
                              dsem[hb]).wait()

    def gat_start(hb, k, b):
        pltpu.make_async_copy(hs_hbm.at[sidx.at[hb].at[k]], rows[b],
                              gsem[b]).start()

    def gat_wait(hb, k, b):
        pltpu.make_async_copy(hs_hbm.at[sidx.at[hb].at[k]], rows[b],
                              gsem[b]).wait()

    def sc_start(hb, k, b):
        pltpu.async_copy(rows[b], acc.at[didx.at[hb].at[k]], csem[b],
                         add=True)

    def sc_wait(hb, k, b):
        pltpu.make_async_copy(rows[b], acc.at[didx.at[hb].at[k]],
                              csem[b]).wait()

    pltpu.sync_copy(hs_hbm.at[pl.ds(c * NP + s * RB, RB)],
                    acc.at[pl.ds(s * RB, RB)])
    grp_start(0, 0)
    plsc.subcore_barrier()

    def outer(h0, _):
        for p in range(2):
            h = h0 * 2 + p
            for k in range(8):
                b = k % 2
                if k >= 2:
                    sc_wait(p, k - 2, b)
                elif p == 1:
                    sc_wait(0, k + 6, b)
                else:
                    @pl.when(h0 >= 1)
                    def _():
                        sc_wait(1, k + 6, b)
                if k == 0:
                    grp_wait(h, p)
                gat_start(p, k, b)
                if k >= 1:
                    gat_wait(p, k - 1, 1 - b)
                    sc_start(p, k - 1, 1 - b)
                elif p == 1:
                    gat_wait(0, 7, 1 - b)
                    sc_start(0, 7, 1 - b)
                else:
                    @pl.when(h0 >= 1)
                    def _():
                        gat_wait(1, 7, 1 - b)
                        sc_start(1, 7, 1 - b)
                if k == 2:
                    if p == 0:
                        grp_start(h + 1, 1)
                    else:
                        @pl.when(h0 < NGRP // 2 - 1)
                        def _():
                            grp_start(h + 1, 0)
        return 0

    lax.fori_loop(0, NGRP // 2, outer, 0)
    gat_wait(1, 7, 1)
    sc_start(1, 7, 1)
    sc_wait(1, 6, 0)
    sc_wait(1, 7, 1)
    plsc.subcore_barrier()
    pltpu.sync_copy(acc.at[pl.ds(s * RB, RB)],
                    out_hbm.at[pl.ds(c * NP + s * RB, RB)])


def _tc1(x_ref, w_ref, deg_ref, out_ref):
    dinv = lax.rsqrt(deg_ref[0, :] + deg_ref[1, :] + 1.0)
    t = jnp.dot(x_ref[...], w_ref[...], preferred_element_type=jnp.float32)
    t = t * dinv[:, None]
    out_ref[0] = t[:, :HH]
    out_ref[1] = t[:, HH:]


def _tc2(a_ref, deg_ref, b_ref, g_ref, be_ref, w_ref, out_ref):
    dinv = lax.rsqrt(deg_ref[0, :] + deg_ref[1, :] + 1.0)
    a = jnp.concatenate([a_ref[0], a_ref[1]], axis=1)
    h = a * dinv[:, None] + b_ref[...]
    h = jnp.maximum(h * (g_ref[...] * BNS) + be_ref[...], 0.0)
    t = jnp.dot(h, w_ref[...], preferred_element_type=jnp.float32)
    t = t * dinv[:, None]
    out_ref[0] = t[:, :HH]
    out_ref[1] = t[:, HH:]


def _tc3(a_ref, deg_ref, b_ref, g_ref, be_ref, w_ref, fb_ref, sec_ref,
         w2_ref, b2_ref, hw1_ref, hb1_ref, hw2_ref, hb2_ref,
         out_ref, tsum_ref, cnt_ref):
    i = pl.program_id(0)
    dinv = lax.rsqrt(deg_ref[0, :] + deg_ref[1, :] + 1.0)
    a = jnp.concatenate([a_ref[0], a_ref[1]], axis=1)
    h = a * dinv[:, None] + b_ref[...]
    h = jnp.maximum(h * (g_ref[...] * BNS) + be_ref[...], 0.0)
    t = jnp.maximum(
        jnp.dot(h, w_ref[...], preferred_element_type=jnp.float32)
        + fb_ref[...], 0.0)
    iot = lax.broadcasted_iota(jnp.int32, (1, S), 1)
    oh = (sec_ref[...] == iot).astype(jnp.float32)
    ts = lax.dot_general(oh, t, (((0,), (0,)), ((), ())),
                         preferred_element_type=jnp.float32)
    cs = jnp.sum(oh, axis=0)[:, None]

    @pl.when(i == 0)
    def _():
        tsum_ref[...] = ts
        cnt_ref[...] = cs

    @pl.when(i > 0)
    def _():
        tsum_ref[...] += ts
        cnt_ref[...] += cs

    @pl.when(i == NB - 1)
    def _():
        cnt = cnt_ref[...]
        meant = tsum_ref[...] / jnp.maximum(cnt, 1.0)
        se = jnp.dot(meant, w2_ref[...], preferred_element_type=jnp.float32)
        se = se + b2_ref[...]
        se = jnp.where(cnt > 0.0, se, 0.0)
        prows = []
        for k in range(S):
            v = jnp.dot(se[k:k + 1, :], hw1_ref[k],
                        preferred_element_type=jnp.float32)
            v = jnp.maximum(v + hb1_ref[k:k + 1, :], 0.0)
            p = (jnp.sum(v * hw2_ref[k], axis=1, keepdims=True)
                 + hb2_ref[k:k + 1, :])
            prows.append(p)
        out_ref[...] = jnp.concatenate(prows, axis=0)


def kernel(x, edge_index, sectors, W0, b0, W1, b1, g0, be0, g1, be1,
           fcW1, fcb1, fcW2, fcb2, HW1, Hb1, HW2, Hb2):
    f32 = jnp.float32
    src, dst = edge_index[0], edge_index[1]
    padn = NP - N
    x_pad = jnp.pad(x, ((0, padn), (0, 0)))
    sec_pad = jnp.pad(sectors, (0, padn), constant_values=S)[:, None]
    pade = EPAD - E
    filler = N + (jnp.arange(pade, dtype=jnp.int32) % padn)
    src_p = jnp.concatenate([src, filler])
    dst_p = jnp.concatenate([dst, filler])
    srcs3 = jnp.concatenate([src_p, src_p + NP]).reshape(32, NG // 8, 8, K)
    dst3 = dst_p.reshape(16, NG // 8, 8, K)
    dst3d = dst_p.reshape(32, NGD, K)

    degpair = _deg(dst3d).reshape(2, NP)

    hs0 = pl.pallas_call(
        _tc1, grid=(NB,),
        in_specs=[pl.BlockSpec((R, D_IN), lambda i: (i, 0)),
                  pl.BlockSpec((D_IN, H), lambda i: (0, 0)),
                  pl.BlockSpec((2, R), lambda i: (0, i))],
        out_specs=pl.BlockSpec((2, R, HH), lambda i: (0, i, 0)),
        out_shape=jax.ShapeDtypeStruct((2, NP, HH), f32),
    )(x_pad, W0, degpair)

    agg0 = _agg(hs0.reshape(2 * NP, HH), srcs3, dst3).reshape(2, NP, HH)

    hs1 = pl.pallas_call(
        _tc2, grid=(NB,),
        in_specs=[pl.BlockSpec((2, R, HH), lambda i: (0, i, 0)),
                  pl.BlockSpec((2, R), lambda i: (0, i)),
                  pl.BlockSpec((1, H), lambda i: (0, 0)),
                  pl.BlockSpec((1, H), lambda i: (0, 0)),
                  pl.BlockSpec((1, H), lambda i: (0, 0)),
                  pl.BlockSpec((H, H), lambda i: (0, 0))],
        out_specs=pl.BlockSpec((2, R, HH), lambda i: (0, i, 0)),
        out_shape=jax.ShapeDtypeStruct((2, NP, HH), f32),
    )(agg0, degpair, b0[None, :], g0[None, :], be0[None, :], W1)

    agg1 = _agg(hs1.reshape(2 * NP, HH), srcs3, dst3).reshape(2, NP, HH)

    preds = pl.pallas_call(
        _tc3, grid=(NB,),
        in_specs=[pl.BlockSpec((2, R, HH), lambda i: (0, i, 0)),
                  pl.BlockSpec((2, R), lambda i: (0, i)),
                  pl.BlockSpec((1, H), lambda i: (0, 0)),
                  pl.BlockSpec((1, H), lambda i: (0, 0)),
                  pl.BlockSpec((1, H), lambda i: (0, 0)),
                  pl.BlockSpec((H, HH), lambda i: (0, 0)),
                  pl.BlockSpec((1, HH), lambda i: (0, 0)),
                  pl.BlockSpec((R, 1), lambda i: (i, 0)),
                  pl.BlockSpec((HH, H), lambda i: (0, 0)),
                  pl.BlockSpec((1, H), lambda i: (0, 0)),
                  pl.BlockSpec((S, H, HH), lambda i: (0, 0, 0)),
                  pl.BlockSpec((S, HH), lambda i: (0, 0)),
                  pl.BlockSpec((S, 1, HH), lambda i: (0, 0, 0)),
                  pl.BlockSpec((S, 1), lambda i: (0, 0))],
        out_specs=pl.BlockSpec((S, 1), lambda i: (0, 0)),
        out_shape=jax.ShapeDtypeStruct((S, 1), f32),
        scratch_shapes=[pltpu.VMEM((S, HH), f32), pltpu.VMEM((S, 1), f32)],
    )(agg1, degpair, b1[None, :], g1[None, :], be1[None, :], fcW1,
      fcb1[None, :], sec_pad, fcW2, fcb2[None, :], HW1, Hb1,
      jnp.transpose(HW2, (0, 2, 1)), Hb2)
    return preds


# speedup vs baseline: 1.0256x; 1.0249x over previous
import functools

import jax
import jax.numpy as jnp
from jax import lax
from jax.experimental import pallas as pl
from jax.experimental.pallas import tpu as pltpu
from jax.experimental.pallas import tpu_sc as plsc

N = 10000
NP = 10240
E = 320000
EPAD = 327680
D_IN = 128
H = 256
HH = 128
S = 11
EPS = 1e-5
BNS = 1.0 / (1.0 + EPS) ** 0.5
K = 128
RB = NP // 16
R = 1024
NB = NP // R
KA = 64
NG = EPAD // 16 // KA
NGD = EPAD // 32 // K

_mesh = plsc.VectorSubcoreMesh(core_axis_name="c", subcore_axis_name="s")


@functools.partial(
    pl.kernel, mesh=_mesh,
    out_type=jax.ShapeDtypeStruct((2 * NP,), jnp.float32),
    scratch_types=[
        pltpu.VMEM((NGD, K), jnp.int32),
        pltpu.VMEM((K,), jnp.float32),
        pltpu.VMEM((RB,), jnp.float32),
        pltpu.VMEM_SHARED((NP,), jnp.float32),
    ],
)
def _deg(dst3_hbm, out_hbm, didx, ones_v, zbuf, acc):
    c = lax.axis_index("c")
    s = lax.axis_index("s")
    w = c * 16 + s

    def fill_ones(i, _):
        ones_v[pl.ds(i * 16, 16)] = jnp.ones((16,), jnp.float32)
        return 0

    lax.fori_loop(0, K // 16, fill_ones, 0)

    def fill_zero(i, _):
        zbuf[pl.ds(i * 16, 16)] = jnp.zeros((16,), jnp.float32)
        return 0

    lax.fori_loop(0, RB // 16, fill_zero, 0)
    pltpu.sync_copy(dst3_hbm.at[w], didx)
    pltpu.sync_copy(zbuf, acc.at[pl.ds(s * RB, RB)])
    plsc.subcore_barrier()

    def chunk(g, _):
        pltpu.sync_copy(ones_v, acc.at[didx.at[g]], add=True)
        return 0

    lax.fori_loop(0, NGD, chunk, 0)
    plsc.subcore_barrier()
    pltpu.sync_copy(acc.at[pl.ds(s * RB, RB)],
                    out_hbm.at[pl.ds(c * NP + s * RB, RB)])


@functools.partial(
    pl.kernel, mesh=_mesh,
    out_type=jax.ShapeDtypeStruct((2 * NP, HH), jnp.float32),
    scratch_types=[
        pltpu.VMEM((2, 16, KA), jnp.int32),
        pltpu.VMEM((2, 16, KA), jnp.int32),
        pltpu.VMEM((KA, HH), jnp.float32),
        pltpu.VMEM((KA, HH), jnp.float32),
        pltpu.VMEM((KA, HH), jnp.float32),
        pltpu.VMEM((KA, HH), jnp.float32),
        pltpu.SemaphoreType.DMA,
        pltpu.SemaphoreType.DMA,
        pltpu.SemaphoreType.DMA,
        pltpu.SemaphoreType.DMA,
        pltpu.SemaphoreType.DMA,
        pltpu.SemaphoreType.DMA,
        pltpu.SemaphoreType.DMA,
        pltpu.SemaphoreType.DMA,
        pltpu.SemaphoreType.DMA,
        pltpu.SemaphoreType.DMA,
        pltpu.SemaphoreType.DMA,
        pltpu.SemaphoreType.DMA,
        pltpu.VMEM_SHARED((NP, HH), jnp.float32),
    ],
)
def _agg(hs_hbm, srcs4_hbm, dst4_hbm, out_hbm, sidx, didx, r0, r1, r2, r3,
         ss0, ss1, ds0, ds1, gs0, gs1, gs2, gs3, cs0, cs1, cs2, cs3, acc):
    c = lax.axis_index("c")
    s = lax.axis_index("s")
    w = c * 16 + s
    rows = [r0, r1, r2, r3]
    ssem = [ss0, ss1]
    dsem = [ds0, ds1]
    gsem = [gs0, gs1, gs2, gs3]
    csem = [cs0, cs1, cs2, cs3]
    NGRP = NG // 16

    def grp_start(h, hb):
        pltpu.make_async_copy(srcs4_hbm.at[w].at[h], sidx.at[hb],
                              ssem[hb]).start()
        pltpu.make_async_copy(dst4_hbm.at[s].at[h], didx.at[hb],
                              dsem[hb]).start()

    def grp_wait(h, hb):
        pltpu.make_async_copy(srcs4_hbm.at[w].at[h], sidx.at[hb],
                              ssem[hb]).wait()
        pltpu.make_async_copy(dst4_hbm.at[s].at[h], didx.at[hb],
                              dsem[hb]).wait()

    def gat_start(hb, k, b):
        pltpu.make_async_copy(hs_hbm.at[sidx.at[hb].at[k]], rows[b],
                              gsem[b]).start()

    def gat_wait(hb, k, b):
        pltpu.make_async_copy(hs_hbm.at[sidx.at[hb].at[k]], rows[b],
                              gsem[b]).wait()

    def sc_start(hb, k, b):
        pltpu.async_copy(rows[b], acc.at[didx.at[hb].at[k]], csem[b],
                         add=True)

    def sc_wait(hb, k, b):
        pltpu.make_async_copy(rows[b], acc.at[didx.at[hb].at[k]],
                              csem[b]).wait()

    pltpu.sync_copy(hs_hbm.at[pl.ds(c * NP + s * RB, RB)],
                    acc.at[pl.ds(s * RB, RB)])
    grp_start(0, 0)
    plsc.subcore_barrier()

    def outer(h0, _):
        for p in range(2):
            h = h0 * 2 + p
            for k in range(16):
                b = k % 4
                b2 = (k + 2) % 4
                if k >= 4:
                    sc_wait(p, k - 4, b)
                elif p == 1:
                    sc_wait(0, k + 12, b)
                else:
                    @pl.when(h0 >= 1)
                    def _():
                        sc_wait(1, k + 12, b)
                if k == 0:
                    grp_wait(h, p)
                gat_start(p, k, b)
                if k >= 2:
                    gat_wait(p, k - 2, b2)
                    sc_start(p, k - 2, b2)
                elif p == 1:
                    gat_wait(0, k + 14, b2)
                    sc_start(0, k + 14, b2)
                else:
                    @pl.when(h0 >= 1)
                    def _():
                        gat_wait(1, k + 14, b2)
                        sc_start(1, k + 14, b2)
                if k == 3:
                    if p == 0:
                        grp_start(h + 1, 1)
                    else:
                        @pl.when(h0 < NGRP // 2 - 1)
                        def _():
                            grp_start(h + 1, 0)
        return 0

    lax.fori_loop(0, NGRP // 2, outer, 0)
    gat_wait(1, 14, 2)
    sc_start(1, 14, 2)
    gat_wait(1, 15, 3)
    sc_start(1, 15, 3)
    sc_wait(1, 12, 0)
    sc_wait(1, 13, 1)
    sc_wait(1, 14, 2)
    sc_wait(1, 15, 3)
    plsc.subcore_barrier()
    pltpu.sync_copy(acc.at[pl.ds(s * RB, RB)],
                    out_hbm.at[pl.ds(c * NP + s * RB, RB)])


def _tc1(x_ref, w_ref, deg_ref, out_ref):
    dinv = lax.rsqrt(deg_ref[0, :] + deg_ref[1, :] + 1.0)
    t = jnp.dot(x_ref[...], w_ref[...], preferred_element_type=jnp.float32)
    t = t * dinv[:, None]
    out_ref[0] = t[:, :HH]
    out_ref[1] = t[:, HH:]


def _tc2(a_ref, deg_ref, b_ref, g_ref, be_ref, w_ref, out_ref):
    dinv = lax.rsqrt(deg_ref[0, :] + deg_ref[1, :] + 1.0)
    a = jnp.concatenate([a_ref[0], a_ref[1]], axis=1)
    h = a * dinv[:, None] + b_ref[...]
    h = jnp.maximum(h * (g_ref[...] * BNS) + be_ref[...], 0.0)
    t = jnp.dot(h, w_ref[...], preferred_element_type=jnp.float32)
    t = t * dinv[:, None]
    out_ref[0] = t[:, :HH]
    out_ref[1] = t[:, HH:]


def _tc3(a_ref, deg_ref, b_ref, g_ref, be_ref, w_ref, fb_ref, sec_ref,
         w2_ref, b2_ref, hw1_ref, hb1_ref, hw2_ref, hb2_ref,
         out_ref, tsum_ref, cnt_ref):
    i = pl.program_id(0)
    dinv = lax.rsqrt(deg_ref[0, :] + deg_ref[1, :] + 1.0)
    a = jnp.concatenate([a_ref[0], a_ref[1]], axis=1)
    h = a * dinv[:, None] + b_ref[...]
    h = jnp.maximum(h * (g_ref[...] * BNS) + be_ref[...], 0.0)
    t = jnp.maximum(
        jnp.dot(h, w_ref[...], preferred_element_type=jnp.float32)
        + fb_ref[...], 0.0)
    iot = lax.broadcasted_iota(jnp.int32, (1, S), 1)
    oh = (sec_ref[...] == iot).astype(jnp.float32)
    ts = lax.dot_general(oh, t, (((0,), (0,)), ((), ())),
                         preferred_element_type=jnp.float32)
    cs = jnp.sum(oh, axis=0)[:, None]

    @pl.when(i == 0)
    def _():
        tsum_ref[...] = ts
        cnt_ref[...] = cs

    @pl.when(i > 0)
    def _():
        tsum_ref[...] += ts
        cnt_ref[...] += cs

    @pl.when(i == NB - 1)
    def _():
        cnt = cnt_ref[...]
        meant = tsum_ref[...] / jnp.maximum(cnt, 1.0)
        se = jnp.dot(meant, w2_ref[...], preferred_element_type=jnp.float32)
        se = se + b2_ref[...]
        se = jnp.where(cnt > 0.0, se, 0.0)
        prows = []
        for k in range(S):
            v = jnp.dot(se[k:k + 1, :], hw1_ref[k],
                        preferred_element_type=jnp.float32)
            v = jnp.maximum(v + hb1_ref[k:k + 1, :], 0.0)
            p = (jnp.sum(v * hw2_ref[k], axis=1, keepdims=True)
                 + hb2_ref[k:k + 1, :])
            prows.append(p)
        out_ref[...] = jnp.concatenate(prows, axis=0)


def kernel(x, edge_index, sectors, W0, b0, W1, b1, g0, be0, g1, be1,
           fcW1, fcb1, fcW2, fcb2, HW1, Hb1, HW2, Hb2):
    f32 = jnp.float32
    src, dst = edge_index[0], edge_index[1]
    padn = NP - N
    x_pad = jnp.pad(x, ((0, padn), (0, 0)))
    sec_pad = jnp.pad(sectors, (0, padn), constant_values=S)[:, None]
    pade = EPAD - E
    filler = N + (jnp.arange(pade, dtype=jnp.int32) % padn)
    src_p = jnp.concatenate([src, filler])
    dst_p = jnp.concatenate([dst, filler])
    srcs3 = jnp.concatenate([src_p, src_p + NP]).reshape(32, NG // 16, 16, KA)
    dst3 = dst_p.reshape(16, NG // 16, 16, KA)
    dst3d = dst_p.reshape(32, NGD, K)

    degpair = _deg(dst3d).reshape(2, NP)

    hs0 = pl.pallas_call(
        _tc1, grid=(NB,),
        in_specs=[pl.BlockSpec((R, D_IN), lambda i: (i, 0)),
                  pl.BlockSpec((D_IN, H), lambda i: (0, 0)),
                  pl.BlockSpec((2, R), lambda i: (0, i))],
        out_specs=pl.BlockSpec((2, R, HH), lambda i: (0, i, 0)),
        out_shape=jax.ShapeDtypeStruct((2, NP, HH), f32),
    )(x_pad, W0, degpair)

    agg0 = _agg(hs0.reshape(2 * NP, HH), srcs3, dst3).reshape(2, NP, HH)

    hs1 = pl.pallas_call(
        _tc2, grid=(NB,),
        in_specs=[pl.BlockSpec((2, R, HH), lambda i: (0, i, 0)),
                  pl.BlockSpec((2, R), lambda i: (0, i)),
                  pl.BlockSpec((1, H), lambda i: (0, 0)),
                  pl.BlockSpec((1, H), lambda i: (0, 0)),
                  pl.BlockSpec((1, H), lambda i: (0, 0)),
                  pl.BlockSpec((H, H), lambda i: (0, 0))],
        out_specs=pl.BlockSpec((2, R, HH), lambda i: (0, i, 0)),
        out_shape=jax.ShapeDtypeStruct((2, NP, HH), f32),
    )(agg0, degpair, b0[None, :], g0[None, :], be0[None, :], W1)

    agg1 = _agg(hs1.reshape(2 * NP, HH), srcs3, dst3).reshape(2, NP, HH)

    preds = pl.pallas_call(
        _tc3, grid=(NB,),
        in_specs=[pl.BlockSpec((2, R, HH), lambda i: (0, i, 0)),
                  pl.BlockSpec((2, R), lambda i: (0, i)),
                  pl.BlockSpec((1, H), lambda i: (0, 0)),
                  pl.BlockSpec((1, H), lambda i: (0, 0)),
                  pl.BlockSpec((1, H), lambda i: (0, 0)),
                  pl.BlockSpec((H, HH), lambda i: (0, 0)),
                  pl.BlockSpec((1, HH), lambda i: (0, 0)),
                  pl.BlockSpec((R, 1), lambda i: (i, 0)),
                  pl.BlockSpec((HH, H), lambda i: (0, 0)),
                  pl.BlockSpec((1, H), lambda i: (0, 0)),
                  pl.BlockSpec((S, H, HH), lambda i: (0, 0, 0)),
                  pl.BlockSpec((S, HH), lambda i: (0, 0)),
                  pl.BlockSpec((S, 1, HH), lambda i: (0, 0, 0)),
                  pl.BlockSpec((S, 1), lambda i: (0, 0))],
        out_specs=pl.BlockSpec((S, 1), lambda i: (0, 0)),
        out_shape=jax.ShapeDtypeStruct((S, 1), f32),
        scratch_shapes=[pltpu.VMEM((S, HH), f32), pltpu.VMEM((S, 1), f32)],
    )(agg1, degpair, b1[None, :], g1[None, :], be1[None, :], fcW1,
      fcb1[None, :], sec_pad, fcW2, fcb2[None, :], HW1, Hb1,
      jnp.transpose(HW2, (0, 2, 1)), Hb2)
    return preds


# speedup vs baseline: 1.0404x; 1.0144x over previous
import functools

import jax
import jax.numpy as jnp
from jax import lax
from jax.experimental import pallas as pl
from jax.experimental.pallas import tpu as pltpu
from jax.experimental.pallas import tpu_sc as plsc

N = 10000
NP = 10240
E = 320000
EPAD = 327680
D_IN = 128
H = 256
HH = 128
S = 11
EPS = 1e-5
BNS = 1.0 / (1.0 + EPS) ** 0.5
K = 128
RB = NP // 16
R = 1024
NB = NP // R
KA = 80
NG = EPAD // 16 // KA
NGD = EPAD // 32 // K

_mesh = plsc.VectorSubcoreMesh(core_axis_name="c", subcore_axis_name="s")


@functools.partial(
    pl.kernel, mesh=_mesh,
    out_type=jax.ShapeDtypeStruct((2 * NP,), jnp.float32),
    scratch_types=[
        pltpu.VMEM((NGD, K), jnp.int32),
        pltpu.VMEM((K,), jnp.float32),
        pltpu.VMEM((RB,), jnp.float32),
        pltpu.VMEM_SHARED((NP,), jnp.float32),
    ],
)
def _deg(dst3_hbm, out_hbm, didx, ones_v, zbuf, acc):
    c = lax.axis_index("c")
    s = lax.axis_index("s")
    w = c * 16 + s

    def fill_ones(i, _):
        ones_v[pl.ds(i * 16, 16)] = jnp.ones((16,), jnp.float32)
        return 0

    lax.fori_loop(0, K // 16, fill_ones, 0)

    def fill_zero(i, _):
        zbuf[pl.ds(i * 16, 16)] = jnp.zeros((16,), jnp.float32)
        return 0

    lax.fori_loop(0, RB // 16, fill_zero, 0)
    pltpu.sync_copy(dst3_hbm.at[w], didx)
    pltpu.sync_copy(zbuf, acc.at[pl.ds(s * RB, RB)])
    plsc.subcore_barrier()

    def chunk(g, _):
        pltpu.sync_copy(ones_v, acc.at[didx.at[g]], add=True)
        return 0

    lax.fori_loop(0, NGD, chunk, 0)
    plsc.subcore_barrier()
    pltpu.sync_copy(acc.at[pl.ds(s * RB, RB)],
                    out_hbm.at[pl.ds(c * NP + s * RB, RB)])


@functools.partial(
    pl.kernel, mesh=_mesh,
    out_type=jax.ShapeDtypeStruct((2 * NP, HH), jnp.float32),
    scratch_types=[
        pltpu.VMEM((2, 16, KA), jnp.int32),
        pltpu.VMEM((2, 16, KA), jnp.int32),
        pltpu.VMEM((KA, HH), jnp.float32),
        pltpu.VMEM((KA, HH), jnp.float32),
        pltpu.VMEM((KA, HH), jnp.float32),
        pltpu.VMEM((KA, HH), jnp.float32),
        pltpu.SemaphoreType.DMA,
        pltpu.SemaphoreType.DMA,
        pltpu.SemaphoreType.DMA,
        pltpu.SemaphoreType.DMA,
        pltpu.SemaphoreType.DMA,
        pltpu.SemaphoreType.DMA,
        pltpu.SemaphoreType.DMA,
        pltpu.SemaphoreType.DMA,
        pltpu.SemaphoreType.DMA,
        pltpu.SemaphoreType.DMA,
        pltpu.SemaphoreType.DMA,
        pltpu.SemaphoreType.DMA,
        pltpu.VMEM_SHARED((NP, HH), jnp.float32),
    ],
)
def _agg(hs_hbm, srcs4_hbm, dst4_hbm, out_hbm, sidx, didx, r0, r1, r2, r3,
         ss0, ss1, ds0, ds1, gs0, gs1, gs2, gs3, cs0, cs1, cs2, cs3, acc):
    c = lax.axis_index("c")
    s = lax.axis_index("s")
    w = c * 16 + s
    rows = [r0, r1, r2, r3]
    ssem = [ss0, ss1]
    dsem = [ds0, ds1]
    gsem = [gs0, gs1, gs2, gs3]
    csem = [cs0, cs1, cs2, cs3]
    NGRP = NG // 16

    def grp_start(h, hb):
        pltpu.make_async_copy(srcs4_hbm.at[w].at[h], sidx.at[hb],
                              ssem[hb]).start()
        pltpu.make_async_copy(dst4_hbm.at[s].at[h], didx.at[hb],
                              dsem[hb]).start()

    def grp_wait(h, hb):
        pltpu.make_async_copy(srcs4_hbm.at[w].at[h], sidx.at[hb],
                              ssem[hb]).wait()
        pltpu.make_async_copy(dst4_hbm.at[s].at[h], didx.at[hb],
                              dsem[hb]).wait()

    def gat_start(hb, k, b):
        pltpu.make_async_copy(hs_hbm.at[sidx.at[hb].at[k]], rows[b],
                              gsem[b]).start()

    def gat_wait(hb, k, b):
        pltpu.make_async_copy(hs_hbm.at[sidx.at[hb].at[k]], rows[b],
                              gsem[b]).wait()

    def sc_start(hb, k, b):
        pltpu.async_copy(rows[b], acc.at[didx.at[hb].at[k]], csem[b],
                         add=True)

    def sc_wait(hb, k, b):
        pltpu.make_async_copy(rows[b], acc.at[didx.at[hb].at[k]],
                              csem[b]).wait()

    pltpu.sync_copy(hs_hbm.at[pl.ds(c * NP + s * RB, RB)],
                    acc.at[pl.ds(s * RB, RB)])
    grp_start(0, 0)
    plsc.subcore_barrier()

    def outer(h0, _):
        for p in range(2):
            h = h0 * 2 + p
            for k in range(16):
                b = k % 4
                b2 = (k + 2) % 4
                if k >= 4:
                    sc_wait(p, k - 4, b)
                elif p == 1:
                    sc_wait(0, k + 12, b)
                else:
                    @pl.when(h0 >= 1)
                    def _():
                        sc_wait(1, k + 12, b)
                if k == 0:
                    grp_wait(h, p)
                gat_start(p, k, b)
                if k >= 2:
                    gat_wait(p, k - 2, b2)
                    sc_start(p, k - 2, b2)
                elif p == 1:
                    gat_wait(0, k + 14, b2)
                    sc_start(0, k + 14, b2)
                else:
                    @pl.when(h0 >= 1)
                    def _():
                        gat_wait(1, k + 14, b2)
                        sc_start(1, k + 14, b2)
                if k == 3:
                    if p == 0:
                        grp_start(h + 1, 1)
                    else:
                        @pl.when(h0 < NGRP // 2 - 1)
                        def _():
                            grp_start(h + 1, 0)
        return 0

    lax.fori_loop(0, NGRP // 2, outer, 0)
    gat_wait(1, 14, 2)
    sc_start(1, 14, 2)
    gat_wait(1, 15, 3)
    sc_start(1, 15, 3)
    sc_wait(1, 12, 0)
    sc_wait(1, 13, 1)
    sc_wait(1, 14, 2)
    sc_wait(1, 15, 3)
    plsc.subcore_barrier()
    pltpu.sync_copy(acc.at[pl.ds(s * RB, RB)],
                    out_hbm.at[pl.ds(c * NP + s * RB, RB)])


def _tc1(x_ref, w_ref, deg_ref, out_ref):
    dinv = lax.rsqrt(deg_ref[0, :] + deg_ref[1, :] + 1.0)
    t = jnp.dot(x_ref[...], w_ref[...], preferred_element_type=jnp.float32)
    t = t * dinv[:, None]
    out_ref[0] = t[:, :HH]
    out_ref[1] = t[:, HH:]


def _tc2(a_ref, deg_ref, b_ref, g_ref, be_ref, w_ref, out_ref):
    dinv = lax.rsqrt(deg_ref[0, :] + deg_ref[1, :] + 1.0)
    a = jnp.concatenate([a_ref[0], a_ref[1]], axis=1)
    h = a * dinv[:, None] + b_ref[...]
    h = jnp.maximum(h * (g_ref[...] * BNS) + be_ref[...], 0.0)
    t = jnp.dot(h, w_ref[...], preferred_element_type=jnp.float32)
    t = t * dinv[:, None]
    out_ref[0] = t[:, :HH]
    out_ref[1] = t[:, HH:]


def _tc3(a_ref, deg_ref, b_ref, g_ref, be_ref, w_ref, fb_ref, sec_ref,
         w2_ref, b2_ref, hw1_ref, hb1_ref, hw2_ref, hb2_ref,
         out_ref, tsum_ref, cnt_ref):
    i = pl.program_id(0)
    dinv = lax.rsqrt(deg_ref[0, :] + deg_ref[1, :] + 1.0)
    a = jnp.concatenate([a_ref[0], a_ref[1]], axis=1)
    h = a * dinv[:, None] + b_ref[...]
    h = jnp.maximum(h * (g_ref[...] * BNS) + be_ref[...], 0.0)
    t = jnp.maximum(
        jnp.dot(h, w_ref[...], preferred_element_type=jnp.float32)
        + fb_ref[...], 0.0)
    iot = lax.broadcasted_iota(jnp.int32, (1, S), 1)
    oh = (sec_ref[...] == iot).astype(jnp.float32)
    ts = lax.dot_general(oh, t, (((0,), (0,)), ((), ())),
                         preferred_element_type=jnp.float32)
    cs = jnp.sum(oh, axis=0)[:, None]

    @pl.when(i == 0)
    def _():
        tsum_ref[...] = ts
        cnt_ref[...] = cs

    @pl.when(i > 0)
    def _():
        tsum_ref[...] += ts
        cnt_ref[...] += cs

    @pl.when(i == NB - 1)
    def _():
        cnt = cnt_ref[...]
        meant = tsum_ref[...] / jnp.maximum(cnt, 1.0)
        se = jnp.dot(meant, w2_ref[...], preferred_element_type=jnp.float32)
        se = se + b2_ref[...]
        se = jnp.where(cnt > 0.0, se, 0.0)
        prows = []
        for k in range(S):
            v = jnp.dot(se[k:k + 1, :], hw1_ref[k],
                        preferred_element_type=jnp.float32)
            v = jnp.maximum(v + hb1_ref[k:k + 1, :], 0.0)
            p = (jnp.sum(v * hw2_ref[k], axis=1, keepdims=True)
                 + hb2_ref[k:k + 1, :])
            prows.append(p)
        out_ref[...] = jnp.concatenate(prows, axis=0)


def kernel(x, edge_index, sectors, W0, b0, W1, b1, g0, be0, g1, be1,
           fcW1, fcb1, fcW2, fcb2, HW1, Hb1, HW2, Hb2):
    f32 = jnp.float32
    src, dst = edge_index[0], edge_index[1]
    padn = NP - N
    x_pad = jnp.pad(x, ((0, padn), (0, 0)))
    sec_pad = jnp.pad(sectors, (0, padn), constant_values=S)[:, None]
    pade = EPAD - E
    filler = N + (jnp.arange(pade, dtype=jnp.int32) % padn)
    src_p = jnp.concatenate([src, filler])
    dst_p = jnp.concatenate([dst, filler])
    srcs3 = jnp.concatenate([src_p, src_p + NP]).reshape(32, NG // 16, 16, KA)
    dst3 = dst_p.reshape(16, NG // 16, 16, KA)
    dst3d = dst_p.reshape(32, NGD, K)

    degpair = _deg(dst3d).reshape(2, NP)

    hs0 = pl.pallas_call(
        _tc1, grid=(NB,),
        in_specs=[pl.BlockSpec((R, D_IN), lambda i: (i, 0)),
                  pl.BlockSpec((D_IN, H), lambda i: (0, 0)),
                  pl.BlockSpec((2, R), lambda i: (0, i))],
        out_specs=pl.BlockSpec((2, R, HH), lambda i: (0, i, 0)),
        out_shape=jax.ShapeDtypeStruct((2, NP, HH), f32),
    )(x_pad, W0, degpair)

    agg0 = _agg(hs0.reshape(2 * NP, HH), srcs3, dst3).reshape(2, NP, HH)

    hs1 = pl.pallas_call(
        _tc2, grid=(NB,),
        in_specs=[pl.BlockSpec((2, R, HH), lambda i: (0, i, 0)),
                  pl.BlockSpec((2, R), lambda i: (0, i)),
                  pl.BlockSpec((1, H), lambda i: (0, 0)),
                  pl.BlockSpec((1, H), lambda i: (0, 0)),
                  pl.BlockSpec((1, H), lambda i: (0, 0)),
                  pl.BlockSpec((H, H), lambda i: (0, 0))],
        out_specs=pl.BlockSpec((2, R, HH), lambda i: (0, i, 0)),
        out_shape=jax.ShapeDtypeStruct((2, NP, HH), f32),
    )(agg0, degpair, b0[None, :], g0[None, :], be0[None, :], W1)

    agg1 = _agg(hs1.reshape(2 * NP, HH), srcs3, dst3).reshape(2, NP, HH)

    preds = pl.pallas_call(
        _tc3, grid=(NB,),
        in_specs=[pl.BlockSpec((2, R, HH), lambda i: (0, i, 0)),
                  pl.BlockSpec((2, R), lambda i: (0, i)),
                  pl.BlockSpec((1, H), lambda i: (0, 0)),
                  pl.BlockSpec((1, H), lambda i: (0, 0)),
                  pl.BlockSpec((1, H), lambda i: (0, 0)),
                  pl.BlockSpec((H, HH), lambda i: (0, 0)),
                  pl.BlockSpec((1, HH), lambda i: (0, 0)),
                  pl.BlockSpec((R, 1), lambda i: (i, 0)),
                  pl.BlockSpec((HH, H), lambda i: (0, 0)),
                  pl.BlockSpec((1, H), lambda i: (0, 0)),
                  pl.BlockSpec((S, H, HH), lambda i: (0, 0, 0)),
                  pl.BlockSpec((S, HH), lambda i: (0, 0)),
                  pl.BlockSpec((S, 1, HH), lambda i: (0, 0, 0)),
                  pl.BlockSpec((S, 1), lambda i: (0, 0))],
        out_specs=pl.BlockSpec((S, 1), lambda i: (0, 0)),
        out_shape=jax.ShapeDtypeStruct((S, 1), f32),
        scratch_shapes=[pltpu.VMEM((S, HH), f32), pltpu.VMEM((S, 1), f32)],
    )(agg1, degpair, b1[None, :], g1[None, :], be1[None, :], fcW1,
      fcb1[None, :], sec_pad, fcW2, fcb2[None, :], HW1, Hb1,
      jnp.transpose(HW2, (0, 2, 1)), Hb2)
    return preds
